# Initial kernel scaffold; baseline (speedup 1.0000x reference)
import jax
import jax.numpy as jnp
from jax.experimental import pallas as pl


def kernel(inputs, emb, W_ih_f, W_hh_f, b_ih_f, b_hh_f, W_ih_b, W_hh_b, b_ih_b, b_hh_b, W_mu, b_mu, W_lv, b_lv):
    raise NotImplementedError("write your pallas kernel here")


# speedup vs baseline: 5.1687x; 5.1687x over previous
import functools

import jax
import jax.numpy as jnp
from jax import lax
from jax.experimental import pallas as pl
from jax.experimental.pallas import tpu as pltpu
from jax.experimental.pallas import tpu_sc as plsc

V = 100000
E = 64
H = 256
L = 64
B = 1024
T = 200

NC = 2
NS = 16
NW = NC * NS
CHUNK = 128
NROWS = (T * B) // CHUNK
ROWS_W = NROWS // NW
PER_W = ROWS_W * CHUNK

@functools.cache
def _make_sc_gather():
    mesh = plsc.VectorSubcoreMesh(
        core_axis_name="c", subcore_axis_name="s", num_cores=NC, num_subcores=NS
    )

    @functools.partial(
        pl.kernel,
        out_type=jax.ShapeDtypeStruct((T * B, E), jnp.float32),
        mesh=mesh,
        scratch_types=[
            pltpu.VMEM((ROWS_W, CHUNK), jnp.int32),
            pltpu.VMEM((CHUNK, E), jnp.float32),
            pltpu.SemaphoreType.DMA,
        ],
        compiler_params=pltpu.CompilerParams(use_tc_tiling_on_sc=False),
    )
    def _sc_gather(emb_hbm, idx_hbm, out_hbm, idx_v, rows_v, sem):
        wid = lax.axis_index("s") * NC + lax.axis_index("c")
        pltpu.sync_copy(idx_hbm.at[wid], idx_v)
        base = wid * PER_W

        def body(j, carry):
            pltpu.async_copy(emb_hbm.at[idx_v.at[j]], rows_v, sem).wait()
            pltpu.sync_copy(rows_v, out_hbm.at[pl.ds(base + j * CHUNK, CHUNK)])
            return carry

        lax.fori_loop(0, ROWS_W, body, 0)

    return _sc_gather


def _rnn_body(x_ref, wih_ref, whh_ref, bih_ref, bhh_ref,
              wihb_ref, bihb_ref, bhhb_ref,
              wmu_ref, bmu_ref, wlv_ref, blv_ref,
              mu_ref, lv_ref, h_scr):
    t = pl.program_id(0)

    @pl.when(t == 0)
    def _():
        h_scr[...] = jnp.zeros_like(h_scr)

    x = x_ref[0]
    h = h_scr[...]
    gi = jnp.dot(x, wih_ref[...], preferred_element_type=jnp.float32) + bih_ref[...]
    gh = jnp.dot(h, whh_ref[...], preferred_element_type=jnp.float32) + bhh_ref[...]
    r = jax.nn.sigmoid(gi[:, :H] + gh[:, :H])
    z = jax.nn.sigmoid(gi[:, H:2 * H] + gh[:, H:2 * H])
    n = jnp.tanh(gi[:, 2 * H:] + r * gh[:, 2 * H:])
    hn = (1.0 - z) * n + z * h
    h_scr[...] = hn

    @pl.when(t == T - 1)
    def _():
        gib = jnp.dot(x, wihb_ref[...], preferred_element_type=jnp.float32) + bihb_ref[...]
        ghb = bhhb_ref[...]
        rb = jax.nn.sigmoid(gib[:, :H] + ghb[:, :H])
        zb = jax.nn.sigmoid(gib[:, H:2 * H] + ghb[:, H:2 * H])
        nb = jnp.tanh(gib[:, 2 * H:] + rb * ghb[:, 2 * H:])
        hb = (1.0 - zb) * nb
        out = hn + hb
        mu_ref[...] = jnp.dot(out, wmu_ref[...], preferred_element_type=jnp.float32) + bmu_ref[...]
        lv_ref[...] = jnp.dot(out, wlv_ref[...], preferred_element_type=jnp.float32) + blv_ref[...]


_FULL2 = lambda t: (0, 0)

_rnn_call = pl.pallas_call(
    _rnn_body,
    grid=(T,),
    in_specs=[
        pl.BlockSpec((1, B, E), lambda t: (t, 0, 0)),
        pl.BlockSpec((E, 3 * H), _FULL2),
        pl.BlockSpec((H, 3 * H), _FULL2),
        pl.BlockSpec((1, 3 * H), _FULL2),
        pl.BlockSpec((1, 3 * H), _FULL2),
        pl.BlockSpec((E, 3 * H), _FULL2),
        pl.BlockSpec((1, 3 * H), _FULL2),
        pl.BlockSpec((1, 3 * H), _FULL2),
        pl.BlockSpec((H, L), _FULL2),
        pl.BlockSpec((1, L), _FULL2),
        pl.BlockSpec((H, L), _FULL2),
        pl.BlockSpec((1, L), _FULL2),
    ],
    out_specs=[pl.BlockSpec((B, L), _FULL2), pl.BlockSpec((B, L), _FULL2)],
    out_shape=[jax.ShapeDtypeStruct((B, L), jnp.float32)] * 2,
    scratch_shapes=[pltpu.VMEM((B, H), jnp.float32)],
)


def kernel(inputs, emb, W_ih_f, W_hh_f, b_ih_f, b_hh_f,
           W_ih_b, W_hh_b, b_ih_b, b_hh_b, W_mu, b_mu, W_lv, b_lv):
    idx = inputs.astype(jnp.int32).T.reshape(NW, ROWS_W, CHUNK)
    x_flat = _make_sc_gather()(emb, idx)
    x3 = x_flat.reshape(T, B, E)
    mu, lv = _rnn_call(
        x3,
        W_ih_f.T, W_hh_f.T, b_ih_f.reshape(1, -1), b_hh_f.reshape(1, -1),
        W_ih_b.T, b_ih_b.reshape(1, -1), b_hh_b.reshape(1, -1),
        W_mu.T, b_mu.reshape(1, -1), W_lv.T, b_lv.reshape(1, -1),
    )
    return (mu, lv)
